# baseline (speedup 1.0000x reference)
import functools

import jax
import jax.numpy as jnp
from jax import lax
from jax.experimental import pallas as pl
from jax.experimental.pallas import tpu as pltpu
from jax.experimental.pallas import tpu_sc as plsc

VOCAB = 100000
HIDDEN = 128
BATCH = 4096
CTX = 20

NUM_CORES = 2
NUM_SUBCORES = 16
NUM_WORKERS = NUM_CORES * NUM_SUBCORES
BPW = BATCH // NUM_WORKERS
LANES = 16
HCHUNKS = HIDDEN // LANES


CB = 16
NCH = BPW // CB


def _sc_gather_sum(idx_t, emb_table):
    mesh = plsc.VectorSubcoreMesh(core_axis_name="c", subcore_axis_name="s")

    @functools.partial(
        pl.kernel,
        out_type=jax.ShapeDtypeStruct((BATCH, HIDDEN), jnp.float32),
        mesh=mesh,
        scratch_types=[
            pltpu.VMEM((CTX, BPW), jnp.int32),
            pltpu.VMEM((2, CTX, CB, HIDDEN), jnp.float32),
            pltpu.VMEM((BPW, HIDDEN), jnp.float32),
            pltpu.SemaphoreType.DMA,
            pltpu.SemaphoreType.DMA,
        ],
    )
    def k(idx_hbm, table_hbm, out_hbm, idx_v, rows_v, acc_v, semA, semB):
        wid = lax.axis_index("s") * NUM_CORES + lax.axis_index("c")
        base = wid * BPW
        pltpu.sync_copy(idx_hbm.at[:, pl.ds(base, BPW)], idx_v)
        sems = (semA, semB)

        def fire(ch, buf):
            return [
                pltpu.async_copy(
                    table_hbm.at[idx_v.at[c, pl.ds(ch * CB, CB)]],
                    rows_v.at[buf, c],
                    sems[buf],
                )
                for c in range(CTX)
            ]

        pending = fire(0, 0)
        for ch in range(NCH):
            buf = ch % 2
            for cp in pending:
                cp.wait()
            if ch + 1 < NCH:
                nxt = fire(ch + 1, (ch + 1) % 2)

            @plsc.parallel_loop(0, CB, 1, unroll=2)
            def b_step(b):
                for h in range(HCHUNKS):
                    sl = pl.ds(h * LANES, LANES)
                    acc = rows_v[buf, 0, b, sl]
                    for c in range(1, CTX):
                        acc = acc + rows_v[buf, c, b, sl]
                    acc_v[ch * CB + b, sl] = acc

            if ch + 1 < NCH:
                pending = nxt

        pltpu.sync_copy(acc_v, out_hbm.at[pl.ds(base, BPW)])

    return k(idx_t, emb_table)


BMV = 800
GV = VOCAB // BMV
NBUF = 3
LAST = GV - 1


def _ring_desc(o_hbm, o_buf, sems, s, v):
    return pltpu.make_async_copy(
        o_buf.at[s],
        o_hbm.at[pl.ds(v * BMV, BMV)],
        sems.at[s],
    )


def _mm_body(w_ref, s_ref, b_ref, o_hbm, o_buf, sems):
    v = pl.program_id(0)
    slot = lax.rem(v, NBUF)

    acc = lax.dot_general(
        w_ref[...].astype(jnp.bfloat16),
        s_ref[...],
        (((1,), (1,)), ((), ())),
        preferred_element_type=jnp.float32,
    ) + b_ref[...]

    @pl.when(v >= NBUF)
    def _():
        _ring_desc(o_hbm, o_buf, sems, slot, v).wait()

    o_buf[slot] = acc
    _ring_desc(o_hbm, o_buf, sems, slot, v).start()

    @pl.when(v == LAST)
    def _():
        for st in range(LAST - NBUF + 1, LAST + 1):
            _ring_desc(o_hbm, o_buf, sems, st % NBUF, v).wait()


def _tc_matmul_t(out_w, summed_bf, out_b2):
    return pl.pallas_call(
        _mm_body,
        grid=(GV,),
        in_specs=[
            pl.BlockSpec((BMV, HIDDEN), lambda v: (v, 0)),
            pl.BlockSpec((BATCH, HIDDEN), lambda v: (0, 0)),
            pl.BlockSpec((BMV, 1), lambda v: (v, 0)),
        ],
        out_specs=pl.BlockSpec(memory_space=pl.ANY),
        out_shape=jax.ShapeDtypeStruct((VOCAB, BATCH), jnp.float32),
        scratch_shapes=[
            pltpu.VMEM((NBUF, BMV, BATCH), jnp.float32),
            pltpu.SemaphoreType.DMA((NBUF,)),
        ],
        compiler_params=pltpu.CompilerParams(
            dimension_semantics=("arbitrary",),
        ),
    )(out_w, summed_bf, out_b2)


def kernel(inputs, emb_table, out_w, out_b):
    idx_t = inputs.T.reshape(CTX, BATCH)
    summed = _sc_gather_sum(idx_t, emb_table)
    logits_t = _tc_matmul_t(
        out_w,
        summed.astype(jnp.bfloat16),
        out_b.reshape(VOCAB, 1),
    )
    return logits_t.T
